# baseline (speedup 1.0000x reference)
import functools

import jax
import jax.numpy as jnp
from jax import lax
from jax.experimental import pallas as pl
from jax.experimental.pallas import tpu as pltpu
from jax.experimental.pallas import tpu_sc as plsc

BATCH = 16384
EMBED_DIM = 64


@functools.cache
def _build(batch, dim):
    info = plsc.get_sparse_core_info()
    nc, ns = info.num_cores, info.num_subcores
    nw = nc * ns
    assert batch % (8 * nw) == 0
    b_per_w = batch // nw
    mesh = plsc.VectorSubcoreMesh(core_axis_name="c", subcore_axis_name="s")

    @functools.partial(
        pl.kernel,
        mesh=mesh,
        out_type=jax.ShapeDtypeStruct((batch, dim), jnp.float32),
        scratch_types=[
            pltpu.VMEM((b_per_w,), jnp.int32),
            pltpu.VMEM((b_per_w, dim), jnp.float32),
            pltpu.SemaphoreType.DMA,
        ],
        compiler_params=pltpu.CompilerParams(use_tc_tiling_on_sc=False),
    )
    def gather_kernel(idx_hbm, table_hbm, out_hbm, idx_v, rows_v, sem):
        wid = lax.axis_index("s") * nc + lax.axis_index("c")
        base = wid * b_per_w
        pltpu.sync_copy(idx_hbm.at[pl.ds(base, b_per_w)], idx_v)
        pltpu.async_copy(table_hbm.at[idx_v], rows_v, sem).wait()
        pltpu.sync_copy(rows_v, out_hbm.at[pl.ds(base, b_per_w)])

    return gather_kernel


def kernel(x, table):
    return _build(x.shape[0], table.shape[1])(x.astype(jnp.int32), table)


# speedup vs baseline: 2.8542x; 2.8542x over previous
import functools

import jax
import jax.numpy as jnp
from jax import lax
from jax.experimental import pallas as pl
from jax.experimental.pallas import tpu as pltpu
from jax.experimental.pallas import tpu_sc as plsc

NBUF = 4


@functools.cache
def _build(batch, rows, dim):
    info = plsc.get_sparse_core_info()
    nc, ns = info.num_cores, info.num_subcores
    nw = nc * ns
    b_per_w = batch // nw
    assert batch % nw == 0
    n_full = (rows // 128) * 128
    tail0 = rows - 128
    mesh = plsc.VectorSubcoreMesh(core_axis_name="c", subcore_axis_name="s")

    @functools.partial(
        pl.kernel,
        mesh=mesh,
        out_type=jax.ShapeDtypeStruct((batch * dim,), jnp.float32),
        scratch_types=[
            pltpu.VMEM((b_per_w,), jnp.int32),
            pltpu.VMEM((NBUF, dim, 128), jnp.float32),
            pltpu.VMEM((b_per_w * dim,), jnp.float32),
            pltpu.SemaphoreType.DMA,
            pltpu.SemaphoreType.DMA,
        ],
        compiler_params=pltpu.CompilerParams(needs_layout_passes=False),
    )
    def gather_kernel(
        idx_hbm, tablet_hbm, tail_hbm, out_hbm, idx_v, stage, ostage, sem, osem
    ):
        wid = lax.axis_index("s") * nc + lax.axis_index("c")
        base = wid * b_per_w
        pltpu.sync_copy(idx_hbm.at[pl.ds(base, b_per_w)], idx_v)
        lanes = lax.iota(jnp.int32, 16)

        def read_idx(e):
            g = lax.mul(lax.div(e, 16), 16)
            v = idx_v[pl.ds(pl.multiple_of(g, 16), 16)]
            return jnp.sum(jnp.where(lanes == lax.rem(e, 16), v, 0))

        def issue(e):
            r = read_idx(e)
            buf = lax.rem(e, NBUF)
            off = lax.mul(lax.div(r, 128), 128)

            @pl.when(off < n_full)
            def _():
                pltpu.async_copy(
                    tablet_hbm.at[:, pl.ds(pl.multiple_of(off, 128), 128)],
                    stage.at[buf],
                    sem,
                )

            @pl.when(off >= n_full)
            def _():
                pltpu.async_copy(tail_hbm, stage.at[buf], sem)

        for e in range(NBUF):
            issue(e)

        def body(e, _):
            buf = lax.rem(e, NBUF)
            pltpu.make_async_copy(
                tablet_hbm.at[:, pl.ds(0, 128)], stage.at[buf], sem
            ).wait()
            r = read_idx(e)
            off = lax.mul(lax.div(r, 128), 128)
            lane = jnp.where(off < n_full, r - off, r - tail0)
            bufv = jnp.full((16,), buf, jnp.int32)
            lanev = jnp.full((16,), lane, jnp.int32)
            for c0 in range(0, dim, 16):
                cv = lanes + c0
                vals = plsc.load_gather(stage, [bufv, cv, lanev])
                ostage[pl.ds(pl.multiple_of(e * dim + c0, 16), 16)] = vals

            @pl.when(e + NBUF < b_per_w)
            def _():
                issue(e + NBUF)

            return ()

        lax.fori_loop(0, b_per_w, body, (), unroll=1)
        pltpu.async_copy(
            ostage, out_hbm.at[pl.ds(base * dim, b_per_w * dim)], osem
        ).wait()

    return gather_kernel


def kernel(x, table):
    rows, dim = table.shape
    batch = x.shape[0]
    tablet = table.T
    tail = lax.slice(tablet, (0, rows - 128), (dim, rows))
    flat = _build(batch, rows, dim)(x.astype(jnp.int32), tablet, tail)
    return flat.reshape(batch, dim)


# speedup vs baseline: 2.9574x; 1.0362x over previous
import functools

import jax
import jax.numpy as jnp
from jax import lax
from jax.experimental import pallas as pl
from jax.experimental.pallas import tpu as pltpu
from jax.experimental.pallas import tpu_sc as plsc

NBUF = 8


@functools.cache
def _build(batch, rows, dim):
    info = plsc.get_sparse_core_info()
    nc, ns = info.num_cores, info.num_subcores
    nw = nc * ns
    b_per_w = batch // nw
    assert batch % nw == 0
    n_full = (rows // 128) * 128
    tail0 = rows - 128
    mesh = plsc.VectorSubcoreMesh(core_axis_name="c", subcore_axis_name="s")

    @functools.partial(
        pl.kernel,
        mesh=mesh,
        out_type=jax.ShapeDtypeStruct((batch * dim,), jnp.float32),
        scratch_types=[
            pltpu.VMEM((b_per_w,), jnp.int32),
            pltpu.VMEM((NBUF, dim, 128), jnp.float32),
            pltpu.VMEM((b_per_w * dim,), jnp.float32),
            pltpu.SemaphoreType.DMA,
            pltpu.SemaphoreType.DMA,
        ],
        compiler_params=pltpu.CompilerParams(needs_layout_passes=False),
    )
    def gather_kernel(
        idx_hbm, tablet_hbm, tail_hbm, out_hbm, idx_v, stage, ostage, sem, osem
    ):
        wid = lax.axis_index("s") * nc + lax.axis_index("c")
        base = wid * b_per_w
        pltpu.sync_copy(idx_hbm.at[pl.ds(base, b_per_w)], idx_v)
        lanes = lax.iota(jnp.int32, 16)

        def read_idx(e):
            g = lax.mul(lax.div(e, 16), 16)
            v = idx_v[pl.ds(pl.multiple_of(g, 16), 16)]
            return jnp.sum(jnp.where(lanes == lax.rem(e, 16), v, 0))

        def issue(e):
            r = read_idx(e)
            buf = lax.rem(e, NBUF)
            off = lax.mul(lax.div(r, 128), 128)

            @pl.when(off < n_full)
            def _():
                pltpu.async_copy(
                    tablet_hbm.at[:, pl.ds(pl.multiple_of(off, 128), 128)],
                    stage.at[buf],
                    sem,
                )

            @pl.when(off >= n_full)
            def _():
                pltpu.async_copy(tail_hbm, stage.at[buf], sem)

        for e in range(NBUF):
            issue(e)

        def body(e, _):
            buf = lax.rem(e, NBUF)
            pltpu.make_async_copy(
                tablet_hbm.at[:, pl.ds(0, 128)], stage.at[buf], sem
            ).wait()
            r = read_idx(e)
            off = lax.mul(lax.div(r, 128), 128)
            lane = jnp.where(off < n_full, r - off, r - tail0)
            bufv = jnp.full((16,), buf, jnp.int32)
            lanev = jnp.full((16,), lane, jnp.int32)
            for c0 in range(0, dim, 16):
                cv = lanes + c0
                vals = plsc.load_gather(stage, [bufv, cv, lanev])
                ostage[pl.ds(pl.multiple_of(e * dim + c0, 16), 16)] = vals

            @pl.when(e + NBUF < b_per_w)
            def _():
                issue(e + NBUF)

            return ()

        lax.fori_loop(0, b_per_w, body, (), unroll=1)
        pltpu.async_copy(
            ostage, out_hbm.at[pl.ds(base * dim, b_per_w * dim)], osem
        ).wait()

    return gather_kernel


def kernel(x, table):
    rows, dim = table.shape
    batch = x.shape[0]
    tablet = table.T
    tail = lax.slice(tablet, (0, rows - 128), (dim, rows))
    flat = _build(batch, rows, dim)(x.astype(jnp.int32), tablet, tail)
    return flat.reshape(batch, dim)


# speedup vs baseline: 4.2660x; 1.4425x over previous
import functools

import jax
import jax.numpy as jnp
from jax import lax
from jax.experimental import pallas as pl
from jax.experimental.pallas import tpu as pltpu
from jax.experimental.pallas import tpu_sc as plsc

NBUF = 8
NSLOT = 64


@functools.cache
def _build(batch, rows, dim):
    info = plsc.get_sparse_core_info()
    nc, ns = info.num_cores, info.num_subcores
    nw = nc * ns
    nblk = (rows + 127) // 128
    bpw = (nblk + nw - 1) // nw
    n_full = (rows // 128) * 128
    tail0 = rows - 128
    nvec = batch // 16
    mesh = plsc.VectorSubcoreMesh(core_axis_name="c", subcore_axis_name="s")

    @functools.partial(
        pl.kernel,
        mesh=mesh,
        out_type=jax.ShapeDtypeStruct((batch * dim,), jnp.float32),
        scratch_types=[
            pltpu.VMEM((batch,), jnp.int32),
            pltpu.VMEM((batch,), jnp.int32),
            pltpu.VMEM((batch,), jnp.int32),
            pltpu.VMEM((256,), jnp.int32),
            pltpu.VMEM((256,), jnp.int32),
            pltpu.VMEM((256,), jnp.int32),
            pltpu.VMEM((272,), jnp.int32),
            pltpu.VMEM((272,), jnp.int32),
            pltpu.VMEM((272,), jnp.int32),
            pltpu.VMEM((NBUF, dim, 128), jnp.float32),
            pltpu.VMEM((NSLOT * dim,), jnp.float32),
            pltpu.SemaphoreType.DMA,
            pltpu.SemaphoreType.DMA,
        ],
        compiler_params=pltpu.CompilerParams(needs_layout_passes=False),
    )
    def gather_kernel(
        idx_hbm,
        tablet_hbm,
        tail_hbm,
        out_hbm,
        xbuf,
        sorted_r,
        sorted_pos,
        hist,
        offs,
        woffs,
        blkids,
        bstart,
        bend,
        stage,
        oslots,
        sem,
        osem,
    ):
        wid = lax.axis_index("s") * nc + lax.axis_index("c")
        lo = wid * bpw
        hi = lo + bpw
        lanes = lax.iota(jnp.int32, 16)
        zeros = jnp.zeros((16,), jnp.int32)

        pltpu.sync_copy(idx_hbm, xbuf)

        def rd16(ref, g):
            return ref[pl.ds(pl.multiple_of(lax.mul(g, 16), 16), 16)]

        def extract(ref, i):
            v = rd16(ref, lax.div(i, 16))
            return jnp.sum(jnp.where(lanes == lax.rem(i, 16), v, 0))

        for i in range(16):
            hist[pl.ds(i * 16, 16)] = zeros

        def hist_body(v, _):
            xv = rd16(xbuf, v)
            blk = lax.shift_right_logical(xv, 7)
            mine = (blk >= lo) & (blk < hi)
            lblk = blk - lo
            cnt, last = plsc.scan_count(lblk, mine)
            plsc.addupdate_scatter(hist, [lblk], cnt, mask=mine & last)
            return ()

        lax.fori_loop(0, nvec, hist_body, (), unroll=2)

        def scan_body(i, carry):
            h = rd16(hist, i)
            s = plsc.cumsum(h)
            offs[pl.ds(pl.multiple_of(lax.mul(i, 16), 16), 16)] = s - h + carry
            return carry + jnp.sum(h)

        lax.fori_loop(0, 16, scan_body, jnp.int32(0), unroll=2)
        for i in range(16):
            woffs[pl.ds(i * 16, 16)] = offs[pl.ds(i * 16, 16)]

        def perm_body(v, _):
            xv = rd16(xbuf, v)
            blk = lax.shift_right_logical(xv, 7)
            mine = (blk >= lo) & (blk < hi)
            lblk = blk - lo
            cur = plsc.load_gather(woffs, [lblk], mask=mine)
            cnt, last = plsc.scan_count(lblk, mine)
            dst = cur + cnt - 1
            posv = lanes + lax.mul(v, 16)
            plsc.store_scatter(sorted_r, [dst], xv, mask=mine)
            plsc.store_scatter(sorted_pos, [dst], posv, mask=mine)
            plsc.addupdate_scatter(woffs, [lblk], cnt, mask=mine & last)
            return ()

        lax.fori_loop(0, nvec, perm_body, (), unroll=2)

        def compact_body2(i, nb):
            h = rd16(hist, i)
            o = rd16(offs, i)
            nonempty = h > 0
            ids = lanes + lax.mul(i, 16) + lo
            plsc.store_compressed(blkids.at[pl.ds(nb, 16)], ids, mask=nonempty)
            plsc.store_compressed(bstart.at[pl.ds(nb, 16)], o, mask=nonempty)
            plsc.store_compressed(bend.at[pl.ds(nb, 16)], o + h, mask=nonempty)
            return nb + jnp.sum(jnp.where(nonempty, 1, 0))

        nblocks = lax.fori_loop(0, 16, compact_body2, jnp.int32(0), unroll=2)

        def issue(j):
            blkid = extract(blkids, j)
            off = lax.mul(blkid, 128)
            buf = lax.rem(j, NBUF)

            @pl.when(off < n_full)
            def _():
                pltpu.async_copy(
                    tablet_hbm.at[:, pl.ds(pl.multiple_of(off, 128), 128)],
                    stage.at[buf],
                    sem,
                )

            @pl.when(off >= n_full)
            def _():
                pltpu.async_copy(tail_hbm, stage.at[buf], sem)

        for j in range(NBUF):

            @pl.when(j < nblocks)
            def _():
                issue(j)

        def sweep_body(j, ocnt):
            buf = lax.rem(j, NBUF)
            pltpu.make_async_copy(
                tablet_hbm.at[:, pl.ds(0, 128)], stage.at[buf], sem
            ).wait()
            blkid = extract(blkids, j)
            off = lax.mul(blkid, 128)
            start = extract(bstart, j)
            end = extract(bend, j)
            bufv = jnp.full((16,), buf, jnp.int32)

            def entry_body(e, oc):
                r = extract(sorted_r, e)
                pos = extract(sorted_pos, e)
                lane = jnp.where(off < n_full, r - off, r - tail0)
                lanev = jnp.full((16,), lane, jnp.int32)
                slot = lax.rem(oc, NSLOT)

                @pl.when(oc >= NSLOT)
                def _():
                    pltpu.make_async_copy(
                        oslots.at[pl.ds(0, dim)],
                        out_hbm.at[pl.ds(0, dim)],
                        osem,
                    ).wait()

                for c0 in range(0, dim, 16):
                    vals = plsc.load_gather(stage, [bufv, lanes + c0, lanev])
                    oslots[
                        pl.ds(pl.multiple_of(slot * dim + c0, 16), 16)
                    ] = vals
                pltpu.async_copy(
                    oslots.at[pl.ds(pl.multiple_of(slot * dim, 16), dim)],
                    out_hbm.at[pl.ds(lax.mul(pos, dim), dim)],
                    osem,
                )
                return oc + 1

            ocnt = lax.fori_loop(start, end, entry_body, ocnt)

            @pl.when(j + NBUF < nblocks)
            def _():
                issue(j + NBUF)

            return ocnt

        ocnt = lax.fori_loop(0, nblocks, sweep_body, jnp.int32(0))

        ndrain = jnp.where(ocnt < NSLOT, ocnt, NSLOT)

        def drain_body(i, _):
            pltpu.make_async_copy(
                oslots.at[pl.ds(0, dim)], out_hbm.at[pl.ds(0, dim)], osem
            ).wait()
            return ()

        lax.fori_loop(0, ndrain, drain_body, ())

    return gather_kernel


def kernel(x, table):
    rows, dim = table.shape
    batch = x.shape[0]
    tablet = table.T
    tail = lax.slice(tablet, (0, rows - 128), (dim, rows))
    flat = _build(batch, rows, dim)(x.astype(jnp.int32), tablet, tail)
    return flat.reshape(batch, dim)
